# initial kernel scaffold (unmeasured)
import jax
import jax.numpy as jnp
from jax import lax
from jax.experimental import pallas as pl
from jax.experimental.pallas import tpu as pltpu

N_DEV = 16


def kernel(x, w_mat):
    m_glob, k_loc = x.shape
    k_glob, n = w_mat.shape
    m_loc = m_glob // N_DEV

    def body(x_ref, w_ref, out_ref, xrow_ref, send_sems, recv_sems):
        me = lax.axis_index("i")

        xrow_ref[:, pl.ds(me * k_loc, k_loc)] = x_ref[pl.ds(me * m_loc, m_loc), :]

        sends = []
        for d in range(1, N_DEV):
            j = lax.rem(me + d, N_DEV)
            rdma = pltpu.make_async_remote_copy(
                src_ref=x_ref.at[pl.ds(j * m_loc, m_loc), :],
                dst_ref=xrow_ref.at[:, pl.ds(me * k_loc, k_loc)],
                send_sem=send_sems.at[d],
                recv_sem=recv_sems.at[d],
                device_id=(j,),
                device_id_type=pl.DeviceIdType.MESH,
            )
            rdma.start()
            sends.append(rdma)

        for d in range(1, N_DEV):
            recv = pltpu.make_async_remote_copy(
                src_ref=x_ref.at[pl.ds(0, m_loc), :],
                dst_ref=xrow_ref.at[:, pl.ds(0, k_loc)],
                send_sem=send_sems.at[d],
                recv_sem=recv_sems.at[d],
                device_id=(me,),
                device_id_type=pl.DeviceIdType.MESH,
            )
            recv.wait_recv()

        acc = jnp.dot(xrow_ref[:, :], w_ref[:, :],
                      preferred_element_type=jnp.float32)
        out_ref[:, :] = jnp.maximum(acc, 0.0)

        for rdma in sends:
            rdma.wait_send()

    return pl.pallas_call(
        body,
        out_shape=jax.ShapeDtypeStruct((m_loc, n), jnp.float32),
        in_specs=[
            pl.BlockSpec(memory_space=pltpu.VMEM),
            pl.BlockSpec(memory_space=pltpu.VMEM),
        ],
        out_specs=pl.BlockSpec(memory_space=pltpu.VMEM),
        scratch_shapes=[
            pltpu.VMEM((m_loc, k_glob), x.dtype),
            pltpu.SemaphoreType.DMA((N_DEV,)),
            pltpu.SemaphoreType.DMA((N_DEV,)),
        ],
    )(x, w_mat)


# baseline (device time: 50916 ns/iter reference)
import jax
import jax.numpy as jnp
from jax import lax
from jax.experimental import pallas as pl
from jax.experimental.pallas import tpu as pltpu

N_DEV = 16


def kernel(x, w_mat):
    m_glob, k_loc = x.shape
    k_glob, n = w_mat.shape
    m_loc = m_glob // N_DEV

    def body(x_ref, w_ref, out_ref, x16_ref, w16_ref, xrow_ref,
             send_sems, recv_sems):
        me = lax.axis_index("i")

        x16_ref[:, :] = x_ref[:, :].astype(jnp.bfloat16)

        xrow_ref[:, pl.ds(me * k_loc, k_loc)] = x16_ref[pl.ds(me * m_loc, m_loc), :]

        sends = []
        for d in range(1, N_DEV):
            j = lax.rem(me + d, N_DEV)
            rdma = pltpu.make_async_remote_copy(
                src_ref=x16_ref.at[pl.ds(j * m_loc, m_loc), :],
                dst_ref=xrow_ref.at[:, pl.ds(me * k_loc, k_loc)],
                send_sem=send_sems.at[d],
                recv_sem=recv_sems.at[d],
                device_id=(j,),
                device_id_type=pl.DeviceIdType.MESH,
            )
            rdma.start()
            sends.append(rdma)

        for d in range(1, N_DEV):
            recv = pltpu.make_async_remote_copy(
                src_ref=x16_ref.at[pl.ds(0, m_loc), :],
                dst_ref=xrow_ref.at[:, pl.ds(0, k_loc)],
                send_sem=send_sems.at[d],
                recv_sem=recv_sems.at[d],
                device_id=(me,),
                device_id_type=pl.DeviceIdType.MESH,
            )
            recv.wait_recv()

        w16_ref[:, :] = w_ref[:, :].astype(jnp.bfloat16)
        acc = jnp.dot(xrow_ref[:, :], w16_ref[:, :],
                      preferred_element_type=jnp.float32)
        out_ref[:, :] = jnp.maximum(acc, 0.0)

        for rdma in sends:
            rdma.wait_send()

    return pl.pallas_call(
        body,
        out_shape=jax.ShapeDtypeStruct((m_loc, n), jnp.float32),
        in_specs=[
            pl.BlockSpec(memory_space=pltpu.VMEM),
            pl.BlockSpec(memory_space=pltpu.VMEM),
        ],
        out_specs=pl.BlockSpec(memory_space=pltpu.VMEM),
        scratch_shapes=[
            pltpu.VMEM((m_glob, k_loc), jnp.bfloat16),
            pltpu.VMEM((k_glob, n), jnp.bfloat16),
            pltpu.VMEM((m_loc, k_glob), jnp.bfloat16),
            pltpu.SemaphoreType.DMA((N_DEV,)),
            pltpu.SemaphoreType.DMA((N_DEV,)),
        ],
        compiler_params=pltpu.CompilerParams(
            vmem_limit_bytes=100 * 1024 * 1024,
        ),
    )(x, w_mat)


# device time: 34598 ns/iter; 1.4716x vs baseline; 1.4716x over previous
import jax
import jax.numpy as jnp
from jax import lax
from jax.experimental import pallas as pl
from jax.experimental.pallas import tpu as pltpu

N_DEV = 16


def kernel(x, w_mat):
    m_glob, k_loc = x.shape
    k_glob, n = w_mat.shape
    m_loc = m_glob // N_DEV
    kb = k_glob // N_DEV

    def body(x_ref, w_hbm, out_ref, x16_ref, xrow_ref, wbuf_ref, acc_ref,
             send_sems, recv_sems, wsems):
        me = lax.axis_index("i")

        ks = [lax.rem(me + (N_DEV - t) % N_DEV, N_DEV) for t in range(N_DEV)]

        wdmas = []
        for t in range(N_DEV):
            dma = pltpu.make_async_copy(
                w_hbm.at[pl.ds(ks[t] * kb, kb), :],
                wbuf_ref.at[t],
                wsems.at[t],
            )
            dma.start()
            wdmas.append(dma)

        x16_ref[:, :] = x_ref[:, :].astype(jnp.bfloat16)
        xrow_ref[:, pl.ds(me * kb, kb)] = x16_ref[pl.ds(me * m_loc, m_loc), :]

        barrier = pltpu.get_barrier_semaphore()
        for d in range(1, N_DEV):
            j = lax.rem(me + d, N_DEV)
            pl.semaphore_signal(barrier, inc=1, device_id=(j,),
                                device_id_type=pl.DeviceIdType.MESH)
        pl.semaphore_wait(barrier, N_DEV - 1)

        sends = []
        for d in range(1, N_DEV):
            j = lax.rem(me + d, N_DEV)
            rdma = pltpu.make_async_remote_copy(
                src_ref=x16_ref.at[pl.ds(j * m_loc, m_loc), :],
                dst_ref=xrow_ref.at[:, pl.ds(me * kb, kb)],
                send_sem=send_sems.at[d],
                recv_sem=recv_sems.at[d],
                device_id=(j,),
                device_id_type=pl.DeviceIdType.MESH,
            )
            rdma.start()
            sends.append(rdma)

        for t in range(N_DEV):
            wdmas[t].wait()
            if t > 0:
                recv = pltpu.make_async_remote_copy(
                    src_ref=x16_ref.at[pl.ds(0, m_loc), :],
                    dst_ref=xrow_ref.at[:, pl.ds(0, kb)],
                    send_sem=send_sems.at[t],
                    recv_sem=recv_sems.at[t],
                    device_id=(me,),
                    device_id_type=pl.DeviceIdType.MESH,
                )
                recv.wait_recv()
            xb = xrow_ref[:, pl.ds(ks[t] * kb, kb)]
            wb = wbuf_ref[t].astype(jnp.bfloat16)
            part = jnp.dot(xb, wb, preferred_element_type=jnp.float32)
            if t == 0:
                acc_ref[:, :] = part
            else:
                acc_ref[:, :] += part

        out_ref[:, :] = jnp.maximum(acc_ref[:, :], 0.0)

        for rdma in sends:
            rdma.wait_send()

    return pl.pallas_call(
        body,
        out_shape=jax.ShapeDtypeStruct((m_loc, n), jnp.float32),
        in_specs=[
            pl.BlockSpec(memory_space=pltpu.VMEM),
            pl.BlockSpec(memory_space=pl.ANY),
        ],
        out_specs=pl.BlockSpec(memory_space=pltpu.VMEM),
        scratch_shapes=[
            pltpu.VMEM((m_glob, k_loc), jnp.bfloat16),
            pltpu.VMEM((m_loc, k_glob), jnp.bfloat16),
            pltpu.VMEM((N_DEV, kb, n), jnp.float32),
            pltpu.VMEM((m_loc, n), jnp.float32),
            pltpu.SemaphoreType.DMA((N_DEV,)),
            pltpu.SemaphoreType.DMA((N_DEV,)),
            pltpu.SemaphoreType.DMA((N_DEV,)),
        ],
        compiler_params=pltpu.CompilerParams(
            collective_id=0,
            vmem_limit_bytes=100 * 1024 * 1024,
        ),
    )(x, w_mat)
